# baseline (speedup 1.0000x reference)
import functools

import jax
import jax.numpy as jnp
from jax import lax
from jax.experimental import pallas as pl
from jax.experimental.pallas import tpu as pltpu
from jax.experimental.pallas import tpu_sc as plsc

VOCAB = 1000000
DIM = 32
B = 16384
L = 200

NC, NS = 2, 16
NW = NC * NS

VB = 32768
SPLIT = 16 * VB
TAIL_LO = 30 * VB
SC_SPAN = TAIL_LO - SPLIT
HSPAN = SC_SPAN // NC
PRJ_N = HSPAN // NS
PCH = [(c * 1024, 1024) for c in range(PRJ_N // 1024)]

TAIL_SRC = SPLIT
TAIL_N = 17024
PSH = 1000064
SEG_TC = SPLIT // NS
SEG_SC = SC_SPAN // NS

N_TOK = B * L
PER_W = N_TOK // NW
CHUNK = 4096
N_CHUNKS = PER_W // CHUNK


def _project_body(x_ref, w_ref, b_ref, o_ref):
    o_ref[:] = jnp.sum(x_ref[:] * w_ref[:], axis=0) + b_ref[0, 0]


def _project(table_t, w_col, b2):
    n_blk = SPLIT // VB + 1

    def _in_map(i):
        return (0, jnp.where(i == n_blk - 1, TAIL_LO // VB, i))

    def _out_map(i):
        return (i,)

    return pl.pallas_call(
        _project_body,
        grid=(n_blk,),
        in_specs=[
            pl.BlockSpec((DIM, VB), _in_map),
            pl.BlockSpec((DIM, 1), lambda i: (0, 0)),
            pl.BlockSpec((1, 1), lambda i: (0, 0), memory_space=pltpu.SMEM),
        ],
        out_specs=pl.BlockSpec((VB,), _out_map),
        out_shape=jax.ShapeDtypeStruct((n_blk * VB,), jnp.float32),
    )(table_t, w_col, b2)


def _project_sc_body(tf_hbm, w_hbm, b_hbm, out_hbm,
                     wv, bv, tb0, tb1, ob0, ob1,
                     sw, st0, st1, so0, so1):
    sid = lax.axis_index("s")
    cid = lax.axis_index("c")
    ocol0 = cid * HSPAN + sid * PRJ_N
    col0 = SPLIT + ocol0
    pltpu.async_copy(w_hbm, wv, sw).wait()
    pltpu.async_copy(b_hbm, bv, sw).wait()
    wa = wv[pl.ds(0, 16)]
    wb = wv[pl.ds(16, 16)]
    bs = bv[pl.ds(0, 16)][0]

    def wd(d):
        return wa[d] if d < 16 else wb[d - 16]

    tbs = (tb0, tb1)
    obs = (ob0, ob1)
    sts = (st0, st1)
    sos = (so0, so1)
    n_ch = len(PCH)

    def load_tab(c):
        off, n = PCH[c]
        return [pltpu.async_copy(
                    tf_hbm.at[pl.ds(d * VOCAB + col0 + off, n)],
                    tbs[c % 2].at[pl.ds(d * n, n)], sts[c % 2])
                for d in range(DIM)]

    def compute(c):
        off, n = PCH[c]
        tb = tbs[c % 2]
        ob = obs[c % 2]

        def body(k, carry):
            k16 = k * 16
            acc = tb[pl.ds(k16, 16)] * wd(0)
            for d in range(1, DIM):
                acc = acc + tb[pl.ds(d * n + k16, 16)] * wd(d)
            ob[pl.ds(k16, 16)] = acc + bs
            return carry

        lax.fori_loop(0, n // 16, body, 0)

    def store(c):
        off, n = PCH[c]
        return pltpu.async_copy(
            obs[c % 2], out_hbm.at[pl.ds(ocol0 + off, n)], sos[c % 2])

    h_t = {0: load_tab(0)}
    h_o = {}
    for c in range(n_ch):
        if c + 1 < n_ch:
            h_t[c + 1] = load_tab(c + 1)
        for h in h_t[c]:
            h.wait()
        if c >= 2:
            h_o[c - 2].wait()
        compute(c)
        h_o[c] = store(c)
    h_o[n_ch - 2].wait()
    h_o[n_ch - 1].wait()


_sc_project = functools.partial(
    pl.kernel,
    mesh=plsc.VectorSubcoreMesh(core_axis_name="c", subcore_axis_name="s"),
    out_type=jax.ShapeDtypeStruct((SC_SPAN,), jnp.float32),
    scratch_types=[
        pltpu.VMEM((DIM,), jnp.float32),
        pltpu.VMEM((16,), jnp.float32),
        pltpu.VMEM((DIM * 1024,), jnp.float32),
        pltpu.VMEM((DIM * 1024,), jnp.float32),
        pltpu.VMEM((1024,), jnp.float32),
        pltpu.VMEM((1024,), jnp.float32),
        pltpu.SemaphoreType.DMA,
        pltpu.SemaphoreType.DMA,
        pltpu.SemaphoreType.DMA,
        pltpu.SemaphoreType.DMA,
        pltpu.SemaphoreType.DMA,
    ],
)(_project_sc_body)


def _gather_body(ptc_hbm, psc_hbm, idx_hbm, out_hbm, p_sh,
                 idx_v0, idx_v1, idx_v2, val_v0, val_v1,
                 sem_stage, si0, si1, si2, sg0, sg1, so0, so1):
    sid = lax.axis_index("s")
    wid = sid * NC + lax.axis_index("c")
    base = wid * PER_W
    idxs = (idx_v0, idx_v1, idx_v2)
    vals = (val_v0, val_v1)
    si = (si0, si1, si2)
    sg = (sg0, sg1)
    so = (so0, so1)

    def load_idx(j):
        return pltpu.async_copy(
            idx_hbm.at[pl.ds(base + j * CHUNK, CHUNK)], idxs[j % 3], si[j % 3])

    def gather(j):
        return pltpu.async_copy(p_sh.at[idxs[j % 3]], vals[j % 2], sg[j % 2])

    def store(j):
        return pltpu.async_copy(
            vals[j % 2], out_hbm.at[pl.ds(base + j * CHUNK, CHUNK)], so[j % 2])

    h_s1 = pltpu.async_copy(ptc_hbm.at[pl.ds(sid * SEG_TC, SEG_TC)],
                            p_sh.at[pl.ds(sid * SEG_TC, SEG_TC)], sem_stage)
    h_s2 = pltpu.async_copy(
        psc_hbm.at[pl.ds(sid * SEG_SC, SEG_SC)],
        p_sh.at[pl.ds(SPLIT + sid * SEG_SC, SEG_SC)], sem_stage)
    h_s3 = pltpu.async_copy(ptc_hbm.at[pl.ds(TAIL_SRC, TAIL_N)],
                            p_sh.at[pl.ds(TAIL_LO, TAIL_N)], sem_stage)
    h_i = {j: load_idx(j) for j in range(min(3, N_CHUNKS))}
    h_s1.wait()
    h_s2.wait()
    h_s3.wait()
    plsc.subcore_barrier()
    h_i[0].wait()
    h_g = {0: gather(0)}
    h_o = {}
    for j in range(N_CHUNKS):
        if j + 1 < N_CHUNKS:
            h_i[j + 1].wait()
            if j >= 1:
                h_o[j - 1].wait()
            h_g[j + 1] = gather(j + 1)
        h_g[j].wait()
        if j + 3 < N_CHUNKS:
            h_i[j + 3] = load_idx(j + 3)
        h_o[j] = store(j)
    h_o[N_CHUNKS - 2].wait()
    h_o[N_CHUNKS - 1].wait()


_sc_gather = functools.partial(
    pl.kernel,
    mesh=plsc.VectorSubcoreMesh(core_axis_name="c", subcore_axis_name="s"),
    out_type=jax.ShapeDtypeStruct((N_TOK,), jnp.float32),
    scratch_types=[
        pltpu.VMEM_SHARED((PSH,), jnp.float32),
        pltpu.VMEM((CHUNK,), jnp.int32),
        pltpu.VMEM((CHUNK,), jnp.int32),
        pltpu.VMEM((CHUNK,), jnp.int32),
        pltpu.VMEM((CHUNK,), jnp.float32),
        pltpu.VMEM((CHUNK,), jnp.float32),
        pltpu.SemaphoreType.DMA,
        pltpu.SemaphoreType.DMA,
        pltpu.SemaphoreType.DMA,
        pltpu.SemaphoreType.DMA,
        pltpu.SemaphoreType.DMA,
        pltpu.SemaphoreType.DMA,
        pltpu.SemaphoreType.DMA,
        pltpu.SemaphoreType.DMA,
    ],
)(_gather_body)


def kernel(token_ids, embed_table, W, b):
    table_t = embed_table.T
    w_col = W.reshape(DIM, 1)
    b2 = jnp.broadcast_to(b.astype(jnp.float32), (1, 1))
    ptc = _project(table_t, w_col, b2)
    psc = _sc_project(table_t.reshape(DIM * VOCAB),
                      W.reshape(DIM).astype(jnp.float32),
                      jnp.broadcast_to(b.astype(jnp.float32), (16,)))
    idx = (token_ids.T.astype(jnp.int32)
           .reshape(L // 8, 8, B // 128, 128)
           .swapaxes(1, 2)
           .reshape(N_TOK))
    out = _sc_gather(ptc, psc, idx)
    return (out.reshape(L // 8, B // 128, 8, 128)
            .swapaxes(1, 2)
            .reshape(L, B)
            .T)


# speedup vs baseline: 1.0044x; 1.0044x over previous
import functools

import jax
import jax.numpy as jnp
from jax import lax
from jax.experimental import pallas as pl
from jax.experimental.pallas import tpu as pltpu
from jax.experimental.pallas import tpu_sc as plsc

VOCAB = 1000000
DIM = 32
B = 16384
L = 200

NC, NS = 2, 16
NW = NC * NS

VB = 32768
SPLIT = 16 * VB
TAIL_LO = 30 * VB
SC_SPAN = TAIL_LO - SPLIT
HSPAN = SC_SPAN // NC
PRJ_N = HSPAN // NS
PCH = [(c * 1024, 1024) for c in range(PRJ_N // 1024)]

TAIL_SRC = SPLIT
TAIL_N = 17024
PSH = 1000064
SEG_TC = SPLIT // NS
SEG_SC = SC_SPAN // NS

N_TOK = B * L
PER_W = N_TOK // NW
CHUNK = 4096
N_CHUNKS = PER_W // CHUNK


def _project_body(x_ref, w_ref, b_ref, o_ref):
    o_ref[:] = jnp.sum(x_ref[:] * w_ref[:], axis=0) + b_ref[0, 0]


def _project(table_t, w_col, b2):
    n_blk = SPLIT // VB + 1

    def _in_map(i):
        return (0, jnp.where(i == n_blk - 1, TAIL_LO // VB, i))

    def _out_map(i):
        return (i,)

    return pl.pallas_call(
        _project_body,
        grid=(n_blk,),
        in_specs=[
            pl.BlockSpec((DIM, VB), _in_map),
            pl.BlockSpec((DIM, 1), lambda i: (0, 0)),
            pl.BlockSpec((1, 1), lambda i: (0, 0), memory_space=pltpu.SMEM),
        ],
        out_specs=pl.BlockSpec((VB,), _out_map),
        out_shape=jax.ShapeDtypeStruct((n_blk * VB,), jnp.float32),
    )(table_t, w_col, b2)


def _project_sc_body(tf_hbm, w_hbm, b_hbm, out_hbm,
                     wv, bv, tb0, tb1, ob0, ob1,
                     sw, st0, st1, so0, so1):
    sid = lax.axis_index("s")
    cid = lax.axis_index("c")
    ocol0 = cid * HSPAN + sid * PRJ_N
    col0 = SPLIT + ocol0
    pltpu.async_copy(w_hbm, wv, sw).wait()
    pltpu.async_copy(b_hbm, bv, sw).wait()
    wa = wv[pl.ds(0, 16)]
    wb = wv[pl.ds(16, 16)]
    bs = bv[pl.ds(0, 16)][0]

    def wd(d):
        return wa[d] if d < 16 else wb[d - 16]

    tbs = (tb0, tb1)
    obs = (ob0, ob1)
    sts = (st0, st1)
    sos = (so0, so1)
    n_ch = len(PCH)

    def load_tab(c):
        off, n = PCH[c]
        return [pltpu.async_copy(
                    tf_hbm.at[pl.ds(d * VOCAB + col0 + off, n)],
                    tbs[c % 2].at[pl.ds(d * n, n)], sts[c % 2])
                for d in range(DIM)]

    def compute(c):
        off, n = PCH[c]
        tb = tbs[c % 2]
        ob = obs[c % 2]

        @plsc.parallel_loop(0, n, 16, unroll=4)
        def body(k16):
            acc = tb[pl.ds(k16, 16)] * wd(0)
            for d in range(1, DIM):
                acc = acc + tb[pl.ds(d * n + k16, 16)] * wd(d)
            ob[pl.ds(k16, 16)] = acc + bs

    def store(c):
        off, n = PCH[c]
        return pltpu.async_copy(
            obs[c % 2], out_hbm.at[pl.ds(ocol0 + off, n)], sos[c % 2])

    h_t = {0: load_tab(0)}
    h_o = {}
    for c in range(n_ch):
        if c + 1 < n_ch:
            h_t[c + 1] = load_tab(c + 1)
        for h in h_t[c]:
            h.wait()
        if c >= 2:
            h_o[c - 2].wait()
        compute(c)
        h_o[c] = store(c)
    h_o[n_ch - 2].wait()
    h_o[n_ch - 1].wait()


_sc_project = functools.partial(
    pl.kernel,
    mesh=plsc.VectorSubcoreMesh(core_axis_name="c", subcore_axis_name="s"),
    out_type=jax.ShapeDtypeStruct((SC_SPAN,), jnp.float32),
    scratch_types=[
        pltpu.VMEM((DIM,), jnp.float32),
        pltpu.VMEM((16,), jnp.float32),
        pltpu.VMEM((DIM * 1024,), jnp.float32),
        pltpu.VMEM((DIM * 1024,), jnp.float32),
        pltpu.VMEM((1024,), jnp.float32),
        pltpu.VMEM((1024,), jnp.float32),
        pltpu.SemaphoreType.DMA,
        pltpu.SemaphoreType.DMA,
        pltpu.SemaphoreType.DMA,
        pltpu.SemaphoreType.DMA,
        pltpu.SemaphoreType.DMA,
    ],
)(_project_sc_body)


def _gather_body(ptc_hbm, psc_hbm, idx_hbm, out_hbm, p_sh,
                 idx_v0, idx_v1, idx_v2, val_v0, val_v1,
                 sem_stage, si0, si1, si2, sg0, sg1, so0, so1):
    sid = lax.axis_index("s")
    wid = sid * NC + lax.axis_index("c")
    base = wid * PER_W
    idxs = (idx_v0, idx_v1, idx_v2)
    vals = (val_v0, val_v1)
    si = (si0, si1, si2)
    sg = (sg0, sg1)
    so = (so0, so1)

    def load_idx(j):
        return pltpu.async_copy(
            idx_hbm.at[pl.ds(base + j * CHUNK, CHUNK)], idxs[j % 3], si[j % 3])

    def gather(j):
        return pltpu.async_copy(p_sh.at[idxs[j % 3]], vals[j % 2], sg[j % 2])

    def store(j):
        return pltpu.async_copy(
            vals[j % 2], out_hbm.at[pl.ds(base + j * CHUNK, CHUNK)], so[j % 2])

    h_s1 = pltpu.async_copy(ptc_hbm.at[pl.ds(sid * SEG_TC, SEG_TC)],
                            p_sh.at[pl.ds(sid * SEG_TC, SEG_TC)], sem_stage)
    h_s2 = pltpu.async_copy(
        psc_hbm.at[pl.ds(sid * SEG_SC, SEG_SC)],
        p_sh.at[pl.ds(SPLIT + sid * SEG_SC, SEG_SC)], sem_stage)
    h_s3 = pltpu.async_copy(ptc_hbm.at[pl.ds(TAIL_SRC, TAIL_N)],
                            p_sh.at[pl.ds(TAIL_LO, TAIL_N)], sem_stage)
    h_i = {j: load_idx(j) for j in range(min(3, N_CHUNKS))}
    h_s1.wait()
    h_s2.wait()
    h_s3.wait()
    plsc.subcore_barrier()
    h_i[0].wait()
    h_g = {0: gather(0)}
    h_o = {}
    for j in range(N_CHUNKS):
        if j + 1 < N_CHUNKS:
            h_i[j + 1].wait()
            if j >= 1:
                h_o[j - 1].wait()
            h_g[j + 1] = gather(j + 1)
        h_g[j].wait()
        if j + 3 < N_CHUNKS:
            h_i[j + 3] = load_idx(j + 3)
        h_o[j] = store(j)
    h_o[N_CHUNKS - 2].wait()
    h_o[N_CHUNKS - 1].wait()


_sc_gather = functools.partial(
    pl.kernel,
    mesh=plsc.VectorSubcoreMesh(core_axis_name="c", subcore_axis_name="s"),
    out_type=jax.ShapeDtypeStruct((N_TOK,), jnp.float32),
    scratch_types=[
        pltpu.VMEM_SHARED((PSH,), jnp.float32),
        pltpu.VMEM((CHUNK,), jnp.int32),
        pltpu.VMEM((CHUNK,), jnp.int32),
        pltpu.VMEM((CHUNK,), jnp.int32),
        pltpu.VMEM((CHUNK,), jnp.float32),
        pltpu.VMEM((CHUNK,), jnp.float32),
        pltpu.SemaphoreType.DMA,
        pltpu.SemaphoreType.DMA,
        pltpu.SemaphoreType.DMA,
        pltpu.SemaphoreType.DMA,
        pltpu.SemaphoreType.DMA,
        pltpu.SemaphoreType.DMA,
        pltpu.SemaphoreType.DMA,
        pltpu.SemaphoreType.DMA,
    ],
)(_gather_body)


def kernel(token_ids, embed_table, W, b):
    table_t = embed_table.T
    w_col = W.reshape(DIM, 1)
    b2 = jnp.broadcast_to(b.astype(jnp.float32), (1, 1))
    ptc = _project(table_t, w_col, b2)
    psc = _sc_project(table_t.reshape(DIM * VOCAB),
                      W.reshape(DIM).astype(jnp.float32),
                      jnp.broadcast_to(b.astype(jnp.float32), (16,)))
    idx = (token_ids.T.astype(jnp.int32)
           .reshape(L // 8, 8, B // 128, 128)
           .swapaxes(1, 2)
           .reshape(N_TOK))
    out = _sc_gather(ptc, psc, idx)
    return (out.reshape(L // 8, B // 128, 8, 128)
            .swapaxes(1, 2)
            .reshape(L, B)
            .T)


# speedup vs baseline: 24.7724x; 24.6648x over previous
import functools

import jax
import jax.numpy as jnp
from jax import lax
from jax.experimental import pallas as pl
from jax.experimental.pallas import tpu as pltpu
from jax.experimental.pallas import tpu_sc as plsc

VOCAB = 1000000
DIM = 32
B = 16384
L = 200

VB = 32768
N_VBLK = 31
PV = N_VBLK * VB

NC, NS = 2, 16
NW = NC * NS
N_TOK = B * L
PER_W = N_TOK // NW
CHUNK = 4096
N_CHUNKS = PER_W // CHUNK
SEG = PV // NS


def _project_body(x_ref, w_ref, b_ref, o_ref):
    o_ref[:] = jnp.sum(x_ref[:] * w_ref[:], axis=0) + b_ref[0, 0]


def _project(table_t, w_col, b2):
    return pl.pallas_call(
        _project_body,
        grid=(N_VBLK,),
        in_specs=[
            pl.BlockSpec((DIM, VB), lambda i: (0, i)),
            pl.BlockSpec((DIM, 1), lambda i: (0, 0)),
            pl.BlockSpec((1, 1), lambda i: (0, 0), memory_space=pltpu.SMEM),
        ],
        out_specs=pl.BlockSpec((VB,), lambda i: (i,)),
        out_shape=jax.ShapeDtypeStruct((PV,), jnp.float32),
    )(table_t, w_col, b2)


def _gather_body(p_hbm, idx_hbm, out_hbm, p_sh,
                 idx_v0, idx_v1, idx_v2, val_v0, val_v1,
                 sem_stage, si0, si1, si2, sg0, sg1, so0, so1):
    sid = lax.axis_index("s")
    wid = sid * NC + lax.axis_index("c")
    base = wid * PER_W
    idxs = (idx_v0, idx_v1, idx_v2)
    vals = (val_v0, val_v1)
    si = (si0, si1, si2)
    sg = (sg0, sg1)
    so = (so0, so1)

    def load_idx(j):
        return pltpu.async_copy(
            idx_hbm.at[pl.ds(base + j * CHUNK, CHUNK)], idxs[j % 3], si[j % 3])

    def gather(j):
        return pltpu.async_copy(p_sh.at[idxs[j % 3]], vals[j % 2], sg[j % 2])

    def store(j):
        return pltpu.async_copy(
            vals[j % 2], out_hbm.at[pl.ds(base + j * CHUNK, CHUNK)], so[j % 2])

    h_stage = pltpu.async_copy(p_hbm.at[pl.ds(sid * SEG, SEG)],
                               p_sh.at[pl.ds(sid * SEG, SEG)], sem_stage)
    h_i = {j: load_idx(j) for j in range(min(3, N_CHUNKS))}
    h_stage.wait()
    plsc.subcore_barrier()
    h_i[0].wait()
    h_g = {0: gather(0)}
    h_o = {}
    for j in range(N_CHUNKS):
        if j + 1 < N_CHUNKS:
            h_i[j + 1].wait()
            if j >= 1:
                h_o[j - 1].wait()
            h_g[j + 1] = gather(j + 1)
        h_g[j].wait()
        if j + 3 < N_CHUNKS:
            h_i[j + 3] = load_idx(j + 3)
        h_o[j] = store(j)
    h_o[N_CHUNKS - 2].wait()
    h_o[N_CHUNKS - 1].wait()


_sc_gather = functools.partial(
    pl.kernel,
    mesh=plsc.VectorSubcoreMesh(core_axis_name="c", subcore_axis_name="s"),
    out_type=jax.ShapeDtypeStruct((N_TOK,), jnp.float32),
    scratch_types=[
        pltpu.VMEM_SHARED((PV,), jnp.float32),
        pltpu.VMEM((CHUNK,), jnp.int32),
        pltpu.VMEM((CHUNK,), jnp.int32),
        pltpu.VMEM((CHUNK,), jnp.int32),
        pltpu.VMEM((CHUNK,), jnp.float32),
        pltpu.VMEM((CHUNK,), jnp.float32),
        pltpu.SemaphoreType.DMA,
        pltpu.SemaphoreType.DMA,
        pltpu.SemaphoreType.DMA,
        pltpu.SemaphoreType.DMA,
        pltpu.SemaphoreType.DMA,
        pltpu.SemaphoreType.DMA,
        pltpu.SemaphoreType.DMA,
        pltpu.SemaphoreType.DMA,
    ],
)(_gather_body)


def kernel(token_ids, embed_table, W, b):
    table_t = embed_table.T
    w_col = W.reshape(DIM, 1)
    b2 = jnp.broadcast_to(b.astype(jnp.float32), (1, 1))
    p = _project(table_t, w_col, b2)
    idx = (token_ids.T.astype(jnp.int32)
           .reshape(L // 8, 8, B // 128, 128)
           .swapaxes(1, 2)
           .reshape(N_TOK))
    out = _sc_gather(p, idx)
    return (out.reshape(L // 8, B // 128, 8, 128)
            .swapaxes(1, 2)
            .reshape(L, B)
            .T)
